# Initial kernel scaffold; baseline (speedup 1.0000x reference)
import jax
import jax.numpy as jnp
from jax.experimental import pallas as pl


def kernel(x, edge_index, edge_weight, beta, y_target, W1, b1, W2, b2, W3, b3, Wmu, bmu, Wlv, blv, Wd1, bd1, Wd2, bd2):
    raise NotImplementedError("write your pallas kernel here")


# speedup vs baseline: 3.9813x; 3.9813x over previous
import functools

import jax
import jax.numpy as jnp
from jax import lax
from jax.experimental import pallas as pl
from jax.experimental.pallas import tpu as pltpu
from jax.experimental.pallas import tpu_sc as plsc

N = 10000
NPAD = 10240
E = 160000
NC = 2
NS = 16
NW = NC * NS
EB = 128
EPT = 5120
E_P = NW * EPT
NB = EPT // EB
RPT = NPAD // NS


def _sc_mesh():
    return plsc.VectorSubcoreMesh(core_axis_name="c", subcore_axis_name="s")


def _deg_body(col_hbm, ew_hbm, zeros_hbm, out_hbm, idx_v, w_v, acc):
    c = lax.axis_index("c")
    s = lax.axis_index("s")
    rows = pl.ds(s * RPT, RPT)
    pltpu.sync_copy(zeros_hbm.at[rows], acc.at[rows])
    plsc.subcore_barrier()
    base = (c * NS + s) * EPT

    def batch(b, carry):
        off = base + b * EB
        pltpu.sync_copy(col_hbm.at[pl.ds(off, EB)], idx_v)
        pltpu.sync_copy(ew_hbm.at[pl.ds(off, EB)], w_v)
        pltpu.sync_copy(w_v, acc.at[idx_v], add=True)
        return carry

    lax.fori_loop(0, NB, batch, 0)
    plsc.subcore_barrier()
    pltpu.sync_copy(acc.at[rows], out_hbm.at[c, rows])


_deg_kernel = pl.kernel(
    _deg_body,
    out_type=jax.ShapeDtypeStruct((NC, NPAD), jnp.float32),
    mesh=_sc_mesh(),
    scratch_types=[
        pltpu.VMEM((EB,), jnp.int32),
        pltpu.VMEM((EB,), jnp.float32),
        pltpu.VMEM_SHARED((NPAD,), jnp.float32),
    ],
)


def _coef_body(row_hbm, col_hbm, ew_hbm, dinv_hbm, out_hbm,
               ri_v, ci_v, ew_v, dr_v, dc_v, cf_v):
    c = lax.axis_index("c")
    s = lax.axis_index("s")
    base = (c * NS + s) * EPT

    def batch(b, carry):
        off = base + b * EB
        pltpu.sync_copy(row_hbm.at[pl.ds(off, EB)], ri_v)
        pltpu.sync_copy(col_hbm.at[pl.ds(off, EB)], ci_v)
        pltpu.sync_copy(ew_hbm.at[pl.ds(off, EB)], ew_v)
        pltpu.sync_copy(dinv_hbm.at[ri_v], dr_v)
        pltpu.sync_copy(dinv_hbm.at[ci_v], dc_v)
        for g in range(EB // 16):
            sl = pl.ds(g * 16, 16)
            cf_v[sl] = dr_v[sl] * ew_v[sl] * dc_v[sl]
        pltpu.sync_copy(cf_v, out_hbm.at[pl.ds(off, EB)])
        return carry

    lax.fori_loop(0, NB, batch, 0)


_coef_kernel = pl.kernel(
    _coef_body,
    out_type=jax.ShapeDtypeStruct((E_P,), jnp.float32),
    mesh=_sc_mesh(),
    scratch_types=[
        pltpu.VMEM((EB,), jnp.int32),
        pltpu.VMEM((EB,), jnp.int32),
        pltpu.VMEM((EB,), jnp.float32),
        pltpu.VMEM((EB,), jnp.float32),
        pltpu.VMEM((EB,), jnp.float32),
        pltpu.VMEM((EB,), jnp.float32),
    ],
)


def _make_agg_kernel(nchunk):
    def body(row_hbm, col_hbm, coef_hbm, *rest):
        srcs = rest[:nchunk]
        zeros_hbm = rest[nchunk]
        out_hbm = rest[nchunk + 1]
        ri_v, ci_v, cf_v, rows_v, acc = rest[nchunk + 2:]
        c = lax.axis_index("c")
        s = lax.axis_index("s")
        myrows = pl.ds(s * RPT, RPT)
        base = (c * NS + s) * EPT
        for k in range(nchunk):
            pltpu.sync_copy(zeros_hbm.at[myrows], acc.at[myrows])
            plsc.subcore_barrier()

            def batch(b, carry):
                off = base + b * EB
                pltpu.sync_copy(row_hbm.at[pl.ds(off, EB)], ri_v)
                pltpu.sync_copy(col_hbm.at[pl.ds(off, EB)], ci_v)
                pltpu.sync_copy(coef_hbm.at[pl.ds(off, EB)], cf_v)
                pltpu.sync_copy(srcs[k].at[ri_v], rows_v)

                def scale(g, cc):
                    w16 = cf_v[pl.ds(g * 16, 16)]
                    for i in range(16):
                        e = g * 16 + i
                        w = w16[i]
                        for j in range(8):
                            sl = pl.ds(j * 16, 16)
                            rows_v[e, sl] = rows_v[e, sl] * w
                    return cc

                lax.fori_loop(0, EB // 16, scale, 0)
                pltpu.sync_copy(rows_v, acc.at[ci_v], add=True)
                return carry

            lax.fori_loop(0, NB, batch, 0)
            plsc.subcore_barrier()
            pltpu.sync_copy(acc.at[myrows], out_hbm.at[k, c, myrows])
            plsc.subcore_barrier()

    return pl.kernel(
        body,
        out_type=jax.ShapeDtypeStruct((nchunk, NC, NPAD, 128), jnp.float32),
        mesh=_sc_mesh(),
        scratch_types=[
            pltpu.VMEM((EB,), jnp.int32),
            pltpu.VMEM((EB,), jnp.int32),
            pltpu.VMEM((EB,), jnp.float32),
            pltpu.VMEM((EB, 128), jnp.float32),
            pltpu.VMEM_SHARED((NPAD, 128), jnp.float32),
        ],
    )


_agg1 = _make_agg_kernel(1)
_agg4 = _make_agg_kernel(4)
_agg2 = _make_agg_kernel(2)


def _dinv_body(degp_ref, out_ref):
    deg = degp_ref[0] + degp_ref[1] + 1.0
    out_ref[...] = jnp.where(
        deg > 0, 1.0 / jnp.sqrt(jnp.maximum(deg, 1e-30)), 0.0)


def _dinv_kernel(degp):
    return pl.pallas_call(
        _dinv_body,
        out_shape=jax.ShapeDtypeStruct((NPAD // 128, 128), jnp.float32),
    )(degp.reshape(NC, NPAD // 128, 128))


_BLK = 256
_GRID = NPAD // _BLK


def _stage1_body(s0_ref, s1_ref, x_ref, dinv_ref, w1_ref, b1_ref, w2_ref,
                 out_ref):
    d = dinv_ref[...]
    a = s0_ref[0, 0] + s1_ref[0, 0] + (d * d) * x_ref[...]
    g = jnp.dot(a, w1_ref[...], preferred_element_type=jnp.float32) + b1_ref[...]
    nrm = jnp.sqrt(jnp.sum(g * g, axis=1, keepdims=True))
    h = jnp.maximum(g / jnp.maximum(nrm, 1e-12), 0.0)
    out_ref[...] = jnp.dot(h, w2_ref[...], preferred_element_type=jnp.float32)


def _stage1(s1, x_p, dinv_col, W1, b1, W2):
    return pl.pallas_call(
        _stage1_body,
        grid=(_GRID,),
        in_specs=[
            pl.BlockSpec((1, 1, _BLK, 128), lambda i: (0, 0, i, 0)),
            pl.BlockSpec((1, 1, _BLK, 128), lambda i: (0, 1, i, 0)),
            pl.BlockSpec((_BLK, 128), lambda i: (i, 0)),
            pl.BlockSpec((_BLK, 1), lambda i: (i, 0)),
            pl.BlockSpec((128, 1024), lambda i: (0, 0)),
            pl.BlockSpec((1, 1024), lambda i: (0, 0)),
            pl.BlockSpec((1024, 512), lambda i: (0, 0)),
        ],
        out_specs=pl.BlockSpec((_BLK, 512), lambda i: (i, 0)),
        out_shape=jax.ShapeDtypeStruct((NPAD, 512), jnp.float32),
    )(s1, s1, x_p, dinv_col, W1, b1.reshape(1, -1), W2)


def _stage2_body(s00, s01, s10, s11, s20, s21, s30, s31, t2_ref, dinv_ref,
                 b2_ref, w3_ref, out_ref):
    d = dinv_ref[...]
    agg = jnp.concatenate(
        [s00[0, 0] + s01[0, 0], s10[0, 0] + s11[0, 0],
         s20[0, 0] + s21[0, 0], s30[0, 0] + s31[0, 0]], axis=1)
    g = agg + (d * d) * t2_ref[...] + b2_ref[...]
    nrm = jnp.sqrt(jnp.sum(g * g, axis=1, keepdims=True))
    h = jnp.maximum(g / jnp.maximum(nrm, 1e-12), 0.0)
    out_ref[...] = jnp.dot(h, w3_ref[...], preferred_element_type=jnp.float32)


def _stage2(s2, t2, dinv_col, b2, W3):
    schunk = [pl.BlockSpec((1, 1, _BLK, 128),
                           functools.partial(lambda k, c, i: (k, c, i, 0), k, c))
              for k in range(4) for c in range(2)]
    return pl.pallas_call(
        _stage2_body,
        grid=(_GRID,),
        in_specs=schunk + [
            pl.BlockSpec((_BLK, 512), lambda i: (i, 0)),
            pl.BlockSpec((_BLK, 1), lambda i: (i, 0)),
            pl.BlockSpec((1, 512), lambda i: (0, 0)),
            pl.BlockSpec((512, 256), lambda i: (0, 0)),
        ],
        out_specs=pl.BlockSpec((_BLK, 256), lambda i: (i, 0)),
        out_shape=jax.ShapeDtypeStruct((NPAD, 256), jnp.float32),
    )(*([s2] * 8), t2, dinv_col, b2.reshape(1, -1), W3)


def _head_body(s00, s01, s10, s11, t3_ref, dinv_ref, b3_ref, wmu_ref, bmu_ref,
               wlv_ref, blv_ref, eps_ref, beta_ref,
               mu_ref, lv_ref, zmax_ref, zsum_ref):
    i = pl.program_id(0)
    d = dinv_ref[...]
    agg = jnp.concatenate(
        [s00[0, 0] + s01[0, 0], s10[0, 0] + s11[0, 0]], axis=1)
    g = agg + (d * d) * t3_ref[...] + b3_ref[...]
    nrm = jnp.sqrt(jnp.sum(g * g, axis=1, keepdims=True))
    h = jnp.maximum(g / jnp.maximum(nrm, 1e-12), 0.0)
    mu = jnp.dot(h, wmu_ref[...], preferred_element_type=jnp.float32) + bmu_ref[...]
    lv = jnp.dot(h, wlv_ref[...], preferred_element_type=jnp.float32) + blv_ref[...]
    mu_ref[...] = mu
    lv_ref[...] = lv
    beta = beta_ref[0, 0]
    z = mu + eps_ref[...] * jnp.exp(0.5 * beta * lv)
    rows = i * _BLK + lax.broadcasted_iota(jnp.int32, (_BLK, 1), 0)
    valid = rows < N
    zmax_blk = jnp.max(jnp.where(valid, z, -jnp.inf), axis=0, keepdims=True)
    zsum_blk = jnp.sum(jnp.where(valid, z, 0.0), axis=0, keepdims=True)

    @pl.when(i == 0)
    def _():
        zmax_ref[...] = zmax_blk
        zsum_ref[...] = zsum_blk

    @pl.when(i > 0)
    def _():
        zmax_ref[...] = jnp.maximum(zmax_ref[...], zmax_blk)
        zsum_ref[...] = zsum_ref[...] + zsum_blk


def _head(s3, t3, dinv_col, b3, Wmu, bmu, Wlv, blv, eps_p, beta_arr):
    schunk = [pl.BlockSpec((1, 1, _BLK, 128),
                           functools.partial(lambda k, c, i: (k, c, i, 0), k, c))
              for k in range(2) for c in range(2)]
    return pl.pallas_call(
        _head_body,
        grid=(_GRID,),
        in_specs=schunk + [
            pl.BlockSpec((_BLK, 256), lambda i: (i, 0)),
            pl.BlockSpec((_BLK, 1), lambda i: (i, 0)),
            pl.BlockSpec((1, 256), lambda i: (0, 0)),
            pl.BlockSpec((256, 512), lambda i: (0, 0)),
            pl.BlockSpec((1, 512), lambda i: (0, 0)),
            pl.BlockSpec((256, 512), lambda i: (0, 0)),
            pl.BlockSpec((1, 512), lambda i: (0, 0)),
            pl.BlockSpec((_BLK, 512), lambda i: (i, 0)),
            pl.BlockSpec((1, 1), lambda i: (0, 0)),
        ],
        out_specs=[
            pl.BlockSpec((_BLK, 512), lambda i: (i, 0)),
            pl.BlockSpec((_BLK, 512), lambda i: (i, 0)),
            pl.BlockSpec((1, 512), lambda i: (0, 0)),
            pl.BlockSpec((1, 512), lambda i: (0, 0)),
        ],
        out_shape=[
            jax.ShapeDtypeStruct((N, 512), jnp.float32),
            jax.ShapeDtypeStruct((N, 512), jnp.float32),
            jax.ShapeDtypeStruct((1, 512), jnp.float32),
            jax.ShapeDtypeStruct((1, 512), jnp.float32),
        ],
    )(*([s3] * 4), t3, dinv_col, b3.reshape(1, -1), Wmu, bmu.reshape(1, -1),
      Wlv, blv.reshape(1, -1), eps_p, beta_arr)


def _dec_body(rz_ref, wd1_ref, bd1_ref, wd2_ref, bd2_ref, out_ref):
    h = rz_ref[...] @ wd1_ref[...] + bd1_ref[...]
    h = jnp.maximum(h, 0.0)
    o = h @ wd2_ref[...] + bd2_ref[...]
    out_ref[...] = 1.0 / (1.0 + jnp.exp(-o))


def _decoder(rz_p, Wd1p, bd1p, Wd2p, bd2):
    return pl.pallas_call(
        _dec_body,
        out_shape=jax.ShapeDtypeStruct((1, 256), jnp.float32),
    )(rz_p, Wd1p, bd1p.reshape(1, -1), Wd2p, bd2.reshape(1, -1))


def kernel(x, edge_index, edge_weight, beta, y_target, W1, b1, W2, b2, W3, b3,
           Wmu, bmu, Wlv, blv, Wd1, bd1, Wd2, bd2):
    row = edge_index[0].astype(jnp.int32)
    col = edge_index[1].astype(jnp.int32)
    pad_e = E_P - E
    row_p = jnp.concatenate([row, jnp.full((pad_e,), NPAD - 1, jnp.int32)])
    col_p = jnp.concatenate([col, jnp.full((pad_e,), NPAD - 1, jnp.int32)])
    ew_p = jnp.concatenate([edge_weight, jnp.zeros((pad_e,), jnp.float32)])
    x_p = jnp.pad(x, ((0, NPAD - N), (0, 0)))
    zeros1 = jnp.zeros((NPAD,), jnp.float32)
    zeros128 = jnp.zeros((NPAD, 128), jnp.float32)

    degp = _deg_kernel(col_p, ew_p, zeros1)
    dinv = _dinv_kernel(degp).reshape(NPAD)
    dinv_col = dinv.reshape(NPAD, 1)
    coef = _coef_kernel(row_p, col_p, ew_p, dinv)

    s1 = _agg1(row_p, col_p, coef, x_p, zeros128)
    t2 = _stage1(s1, x_p, dinv_col, W1, b1, W2)

    s2 = _agg4(row_p, col_p, coef,
               t2[:, 0:128], t2[:, 128:256], t2[:, 256:384], t2[:, 384:512],
               zeros128)
    t3 = _stage2(s2, t2, dinv_col, b2, W3)

    s3 = _agg2(row_p, col_p, coef, t3[:, 0:128], t3[:, 128:256], zeros128)

    eps = jax.random.normal(jax.random.key(42), (N, 512), jnp.float32) * 0.01
    eps_p = jnp.pad(eps, ((0, NPAD - N), (0, 0)))
    beta_arr = jnp.asarray(beta, jnp.float32).reshape(1, 1)
    mu, logvar, zmax, zsum = _head(s3, t3, dinv_col, b3, Wmu, bmu, Wlv, blv,
                                   eps_p, beta_arr)

    rz = jnp.concatenate(
        [zmax, zsum * (1.0 / N), y_target.reshape(1, 1).astype(jnp.float32),
         jnp.zeros((1, 7), jnp.float32)], axis=1)
    Wd1p = jnp.pad(Wd1, ((0, 7), (0, 7)))
    bd1p = jnp.pad(bd1, (0, 7))
    Wd2p = jnp.pad(Wd2, ((0, 7), (0, 0)))
    recon = _decoder(rz, Wd1p, bd1p, Wd2p, bd2)
    return (recon, mu, logvar)


# speedup vs baseline: 5.2782x; 1.3258x over previous
import functools

import jax
import jax.numpy as jnp
from jax import lax
from jax.experimental import pallas as pl
from jax.experimental.pallas import tpu as pltpu
from jax.experimental.pallas import tpu_sc as plsc

N = 10000
NPAD = 10240
E = 160000
NC = 2
NS = 16
NW = NC * NS
EB = 128
EPT = 5120
E_P = NW * EPT
NB = EPT // EB
RPT = NPAD // NS


def _sc_mesh():
    return plsc.VectorSubcoreMesh(core_axis_name="c", subcore_axis_name="s")


def _deg_body(col_hbm, ew_hbm, zeros_hbm, out_hbm, ci_all, ew_all, acc):
    c = lax.axis_index("c")
    s = lax.axis_index("s")
    rows = pl.ds(s * RPT, RPT)
    pltpu.sync_copy(zeros_hbm.at[rows], acc.at[rows])
    tb = (c * NS + s) * NB
    pltpu.sync_copy(col_hbm.at[pl.ds(tb, NB)], ci_all)
    pltpu.sync_copy(ew_hbm.at[pl.ds(tb, NB)], ew_all)
    plsc.subcore_barrier()

    def batch(b, carry):
        pltpu.sync_copy(ew_all.at[b], acc.at[ci_all.at[b]], add=True)
        return carry

    lax.fori_loop(0, NB, batch, 0)
    plsc.subcore_barrier()
    pltpu.sync_copy(acc.at[rows], out_hbm.at[c, rows])


_deg_kernel = pl.kernel(
    _deg_body,
    out_type=jax.ShapeDtypeStruct((NC, NPAD), jnp.float32),
    mesh=_sc_mesh(),
    scratch_types=[
        pltpu.VMEM((NB, EB), jnp.int32),
        pltpu.VMEM((NB, EB), jnp.float32),
        pltpu.VMEM_SHARED((NPAD,), jnp.float32),
    ],
)


def _coef_body(row_hbm, col_hbm, ew_hbm, dinv_hbm, out_hbm,
               ri_all, ci_all, ew_all, dr_all, dc_all, cf_all, sem):
    c = lax.axis_index("c")
    s = lax.axis_index("s")
    tb = (c * NS + s) * NB
    pltpu.sync_copy(row_hbm.at[pl.ds(tb, NB)], ri_all)
    pltpu.sync_copy(col_hbm.at[pl.ds(tb, NB)], ci_all)
    pltpu.sync_copy(ew_hbm.at[pl.ds(tb, NB)], ew_all)

    def batch(b, carry):
        d1 = pltpu.async_copy(dinv_hbm.at[ri_all.at[b]], dr_all.at[b], sem)
        d2 = pltpu.async_copy(dinv_hbm.at[ci_all.at[b]], dc_all.at[b], sem)
        d1.wait()
        d2.wait()
        for g in range(EB // 16):
            sl = pl.ds(g * 16, 16)
            cf_all[b, sl] = dr_all[b, sl] * ew_all[b, sl] * dc_all[b, sl]
        return carry

    lax.fori_loop(0, NB, batch, 0)
    pltpu.sync_copy(cf_all, out_hbm.at[pl.ds(tb, NB)])


_coef_kernel = pl.kernel(
    _coef_body,
    out_type=jax.ShapeDtypeStruct((E_P // EB, EB), jnp.float32),
    mesh=_sc_mesh(),
    scratch_types=[
        pltpu.VMEM((NB, EB), jnp.int32),
        pltpu.VMEM((NB, EB), jnp.int32),
        pltpu.VMEM((NB, EB), jnp.float32),
        pltpu.VMEM((NB, EB), jnp.float32),
        pltpu.VMEM((NB, EB), jnp.float32),
        pltpu.VMEM((NB, EB), jnp.float32),
        pltpu.SemaphoreType.DMA,
    ],
)


def _make_agg_kernel(nchunk):
    NBP = NB // 2

    def body(row_hbm, col_hbm, coef_hbm, *rest):
        srcs = rest[:nchunk]
        zeros_hbm = rest[nchunk]
        out_hbm = rest[nchunk + 1]
        (ri_all, ci_all, cf_all, rows0, rows1, acc,
         gsem0, gsem1, ssem0, ssem1) = rest[nchunk + 2:]
        c = lax.axis_index("c")
        s = lax.axis_index("s")
        myrows = pl.ds(s * RPT, RPT)
        tb = (c * NS + s) * NB
        pltpu.sync_copy(row_hbm.at[pl.ds(tb, NB)], ri_all)
        pltpu.sync_copy(col_hbm.at[pl.ds(tb, NB)], ci_all)
        pltpu.sync_copy(coef_hbm.at[pl.ds(tb, NB)], cf_all)

        def scale(buf, b):
            def grp(g, cc):
                w16 = cf_all[b, pl.ds(g * 16, 16)]
                for i in range(16):
                    e = g * 16 + i
                    w = w16[i]
                    for j in range(8):
                        sl = pl.ds(j * 16, 16)
                        buf[e, sl] = buf[e, sl] * w
                return cc

            lax.fori_loop(0, EB // 16, grp, 0)

        for k in range(nchunk):
            src = srcs[k]
            dummy = src.at[pl.ds(0, EB)]
            pltpu.sync_copy(zeros_hbm.at[myrows], acc.at[myrows])
            plsc.subcore_barrier()
            pltpu.async_copy(src.at[ri_all.at[0]], rows0, gsem0)

            def pair(p, carry):
                b0 = 2 * p

                @pl.when(p > 0)
                def _():
                    pltpu.make_async_copy(dummy, rows1, ssem1).wait()

                pltpu.async_copy(src.at[ri_all.at[b0 + 1]], rows1, gsem1)
                pltpu.make_async_copy(dummy, rows0, gsem0).wait()
                scale(rows0, b0)
                pltpu.async_copy(rows0, acc.at[ci_all.at[b0]], ssem0,
                                 add=True)
                pltpu.make_async_copy(dummy, rows1, gsem1).wait()
                scale(rows1, b0 + 1)
                pltpu.async_copy(rows1, acc.at[ci_all.at[b0 + 1]], ssem1,
                                 add=True)

                @pl.when(p + 1 < NBP)
                def _():
                    pltpu.make_async_copy(dummy, rows0, ssem0).wait()
                    pltpu.async_copy(src.at[ri_all.at[b0 + 2]], rows0, gsem0)

                return carry

            lax.fori_loop(0, NBP, pair, 0)
            pltpu.make_async_copy(dummy, rows1, ssem1).wait()
            pltpu.make_async_copy(dummy, rows0, ssem0).wait()
            plsc.subcore_barrier()
            pltpu.sync_copy(acc.at[myrows], out_hbm.at[k, c, myrows])
            plsc.subcore_barrier()

    return pl.kernel(
        body,
        out_type=jax.ShapeDtypeStruct((nchunk, NC, NPAD, 128), jnp.float32),
        mesh=_sc_mesh(),
        scratch_types=[
            pltpu.VMEM((NB, EB), jnp.int32),
            pltpu.VMEM((NB, EB), jnp.int32),
            pltpu.VMEM((NB, EB), jnp.float32),
            pltpu.VMEM((EB, 128), jnp.float32),
            pltpu.VMEM((EB, 128), jnp.float32),
            pltpu.VMEM_SHARED((NPAD, 128), jnp.float32),
            pltpu.SemaphoreType.DMA,
            pltpu.SemaphoreType.DMA,
            pltpu.SemaphoreType.DMA,
            pltpu.SemaphoreType.DMA,
        ],
    )


_agg1 = _make_agg_kernel(1)
_agg4 = _make_agg_kernel(4)
_agg2 = _make_agg_kernel(2)


def _dinv_body(degp_ref, out_ref):
    deg = degp_ref[0] + degp_ref[1] + 1.0
    out_ref[...] = jnp.where(
        deg > 0, 1.0 / jnp.sqrt(jnp.maximum(deg, 1e-30)), 0.0)


def _dinv_kernel(degp):
    return pl.pallas_call(
        _dinv_body,
        out_shape=jax.ShapeDtypeStruct((NPAD // 128, 128), jnp.float32),
    )(degp.reshape(NC, NPAD // 128, 128))


_BLK = 256
_GRID = NPAD // _BLK


def _stage1_body(s0_ref, s1_ref, x_ref, dinv_ref, w1_ref, b1_ref, w2_ref,
                 out_ref):
    d = dinv_ref[...]
    a = s0_ref[0, 0] + s1_ref[0, 0] + (d * d) * x_ref[...]
    g = jnp.dot(a, w1_ref[...], preferred_element_type=jnp.float32) + b1_ref[...]
    nrm = jnp.sqrt(jnp.sum(g * g, axis=1, keepdims=True))
    h = jnp.maximum(g / jnp.maximum(nrm, 1e-12), 0.0)
    out_ref[...] = jnp.dot(h, w2_ref[...], preferred_element_type=jnp.float32)


def _stage1(s1, x_p, dinv_col, W1, b1, W2):
    return pl.pallas_call(
        _stage1_body,
        grid=(_GRID,),
        in_specs=[
            pl.BlockSpec((1, 1, _BLK, 128), lambda i: (0, 0, i, 0)),
            pl.BlockSpec((1, 1, _BLK, 128), lambda i: (0, 1, i, 0)),
            pl.BlockSpec((_BLK, 128), lambda i: (i, 0)),
            pl.BlockSpec((_BLK, 1), lambda i: (i, 0)),
            pl.BlockSpec((128, 1024), lambda i: (0, 0)),
            pl.BlockSpec((1, 1024), lambda i: (0, 0)),
            pl.BlockSpec((1024, 512), lambda i: (0, 0)),
        ],
        out_specs=pl.BlockSpec((_BLK, 512), lambda i: (i, 0)),
        out_shape=jax.ShapeDtypeStruct((NPAD, 512), jnp.float32),
    )(s1, s1, x_p, dinv_col, W1, b1.reshape(1, -1), W2)


def _stage2_body(s00, s01, s10, s11, s20, s21, s30, s31, t2_ref, dinv_ref,
                 b2_ref, w3_ref, out_ref):
    d = dinv_ref[...]
    agg = jnp.concatenate(
        [s00[0, 0] + s01[0, 0], s10[0, 0] + s11[0, 0],
         s20[0, 0] + s21[0, 0], s30[0, 0] + s31[0, 0]], axis=1)
    g = agg + (d * d) * t2_ref[...] + b2_ref[...]
    nrm = jnp.sqrt(jnp.sum(g * g, axis=1, keepdims=True))
    h = jnp.maximum(g / jnp.maximum(nrm, 1e-12), 0.0)
    out_ref[...] = jnp.dot(h, w3_ref[...], preferred_element_type=jnp.float32)


def _stage2(s2, t2, dinv_col, b2, W3):
    schunk = [pl.BlockSpec((1, 1, _BLK, 128),
                           functools.partial(lambda k, c, i: (k, c, i, 0), k, c))
              for k in range(4) for c in range(2)]
    return pl.pallas_call(
        _stage2_body,
        grid=(_GRID,),
        in_specs=schunk + [
            pl.BlockSpec((_BLK, 512), lambda i: (i, 0)),
            pl.BlockSpec((_BLK, 1), lambda i: (i, 0)),
            pl.BlockSpec((1, 512), lambda i: (0, 0)),
            pl.BlockSpec((512, 256), lambda i: (0, 0)),
        ],
        out_specs=pl.BlockSpec((_BLK, 256), lambda i: (i, 0)),
        out_shape=jax.ShapeDtypeStruct((NPAD, 256), jnp.float32),
    )(*([s2] * 8), t2, dinv_col, b2.reshape(1, -1), W3)


def _head_body(s00, s01, s10, s11, t3_ref, dinv_ref, b3_ref, wmu_ref, bmu_ref,
               wlv_ref, blv_ref, eps_ref, beta_ref,
               mu_ref, lv_ref, zmax_ref, zsum_ref):
    i = pl.program_id(0)
    d = dinv_ref[...]
    agg = jnp.concatenate(
        [s00[0, 0] + s01[0, 0], s10[0, 0] + s11[0, 0]], axis=1)
    g = agg + (d * d) * t3_ref[...] + b3_ref[...]
    nrm = jnp.sqrt(jnp.sum(g * g, axis=1, keepdims=True))
    h = jnp.maximum(g / jnp.maximum(nrm, 1e-12), 0.0)
    mu = jnp.dot(h, wmu_ref[...], preferred_element_type=jnp.float32) + bmu_ref[...]
    lv = jnp.dot(h, wlv_ref[...], preferred_element_type=jnp.float32) + blv_ref[...]
    mu_ref[...] = mu
    lv_ref[...] = lv
    beta = beta_ref[0, 0]
    z = mu + eps_ref[...] * jnp.exp(0.5 * beta * lv)
    rows = i * _BLK + lax.broadcasted_iota(jnp.int32, (_BLK, 1), 0)
    valid = rows < N
    zmax_blk = jnp.max(jnp.where(valid, z, -jnp.inf), axis=0, keepdims=True)
    zsum_blk = jnp.sum(jnp.where(valid, z, 0.0), axis=0, keepdims=True)

    @pl.when(i == 0)
    def _():
        zmax_ref[...] = zmax_blk
        zsum_ref[...] = zsum_blk

    @pl.when(i > 0)
    def _():
        zmax_ref[...] = jnp.maximum(zmax_ref[...], zmax_blk)
        zsum_ref[...] = zsum_ref[...] + zsum_blk


def _head(s3, t3, dinv_col, b3, Wmu, bmu, Wlv, blv, eps_p, beta_arr):
    schunk = [pl.BlockSpec((1, 1, _BLK, 128),
                           functools.partial(lambda k, c, i: (k, c, i, 0), k, c))
              for k in range(2) for c in range(2)]
    return pl.pallas_call(
        _head_body,
        grid=(_GRID,),
        in_specs=schunk + [
            pl.BlockSpec((_BLK, 256), lambda i: (i, 0)),
            pl.BlockSpec((_BLK, 1), lambda i: (i, 0)),
            pl.BlockSpec((1, 256), lambda i: (0, 0)),
            pl.BlockSpec((256, 512), lambda i: (0, 0)),
            pl.BlockSpec((1, 512), lambda i: (0, 0)),
            pl.BlockSpec((256, 512), lambda i: (0, 0)),
            pl.BlockSpec((1, 512), lambda i: (0, 0)),
            pl.BlockSpec((_BLK, 512), lambda i: (i, 0)),
            pl.BlockSpec((1, 1), lambda i: (0, 0)),
        ],
        out_specs=[
            pl.BlockSpec((_BLK, 512), lambda i: (i, 0)),
            pl.BlockSpec((_BLK, 512), lambda i: (i, 0)),
            pl.BlockSpec((1, 512), lambda i: (0, 0)),
            pl.BlockSpec((1, 512), lambda i: (0, 0)),
        ],
        out_shape=[
            jax.ShapeDtypeStruct((N, 512), jnp.float32),
            jax.ShapeDtypeStruct((N, 512), jnp.float32),
            jax.ShapeDtypeStruct((1, 512), jnp.float32),
            jax.ShapeDtypeStruct((1, 512), jnp.float32),
        ],
    )(*([s3] * 4), t3, dinv_col, b3.reshape(1, -1), Wmu, bmu.reshape(1, -1),
      Wlv, blv.reshape(1, -1), eps_p, beta_arr)


def _dec_body(rz_ref, wd1_ref, bd1_ref, wd2_ref, bd2_ref, out_ref):
    h = rz_ref[...] @ wd1_ref[...] + bd1_ref[...]
    h = jnp.maximum(h, 0.0)
    o = h @ wd2_ref[...] + bd2_ref[...]
    out_ref[...] = 1.0 / (1.0 + jnp.exp(-o))


def _decoder(rz_p, Wd1p, bd1p, Wd2p, bd2):
    return pl.pallas_call(
        _dec_body,
        out_shape=jax.ShapeDtypeStruct((1, 256), jnp.float32),
    )(rz_p, Wd1p, bd1p.reshape(1, -1), Wd2p, bd2.reshape(1, -1))


def kernel(x, edge_index, edge_weight, beta, y_target, W1, b1, W2, b2, W3, b3,
           Wmu, bmu, Wlv, blv, Wd1, bd1, Wd2, bd2):
    row = edge_index[0].astype(jnp.int32)
    col = edge_index[1].astype(jnp.int32)
    pad_e = E_P - E
    row_p = jnp.concatenate(
        [row, jnp.full((pad_e,), NPAD - 1, jnp.int32)]).reshape(-1, EB)
    col_p = jnp.concatenate(
        [col, jnp.full((pad_e,), NPAD - 1, jnp.int32)]).reshape(-1, EB)
    ew_p = jnp.concatenate(
        [edge_weight, jnp.zeros((pad_e,), jnp.float32)]).reshape(-1, EB)
    x_p = jnp.pad(x, ((0, NPAD - N), (0, 0)))
    zeros1 = jnp.zeros((NPAD,), jnp.float32)
    zeros128 = jnp.zeros((NPAD, 128), jnp.float32)

    degp = _deg_kernel(col_p, ew_p, zeros1)
    dinv = _dinv_kernel(degp).reshape(NPAD)
    dinv_col = dinv.reshape(NPAD, 1)
    coef = _coef_kernel(row_p, col_p, ew_p, dinv)

    s1 = _agg1(row_p, col_p, coef, x_p, zeros128)
    t2 = _stage1(s1, x_p, dinv_col, W1, b1, W2)

    s2 = _agg4(row_p, col_p, coef,
               t2[:, 0:128], t2[:, 128:256], t2[:, 256:384], t2[:, 384:512],
               zeros128)
    t3 = _stage2(s2, t2, dinv_col, b2, W3)

    s3 = _agg2(row_p, col_p, coef, t3[:, 0:128], t3[:, 128:256], zeros128)

    eps = jax.random.normal(jax.random.key(42), (N, 512), jnp.float32) * 0.01
    eps_p = jnp.pad(eps, ((0, NPAD - N), (0, 0)))
    beta_arr = jnp.asarray(beta, jnp.float32).reshape(1, 1)
    mu, logvar, zmax, zsum = _head(s3, t3, dinv_col, b3, Wmu, bmu, Wlv, blv,
                                   eps_p, beta_arr)

    rz = jnp.concatenate(
        [zmax, zsum * (1.0 / N), y_target.reshape(1, 1).astype(jnp.float32),
         jnp.zeros((1, 7), jnp.float32)], axis=1)
    Wd1p = jnp.pad(Wd1, ((0, 7), (0, 7)))
    bd1p = jnp.pad(bd1, (0, 7))
    Wd2p = jnp.pad(Wd2, ((0, 7), (0, 0)))
    recon = _decoder(rz, Wd1p, bd1p, Wd2p, bd2)
    return (recon, mu, logvar)
